# Initial kernel scaffold; baseline (speedup 1.0000x reference)
import jax
import jax.numpy as jnp
from jax.experimental import pallas as pl


def kernel(hidden_states, position_ids, rms1_w, q_w, q_b, k_w, k_b, v_w, v_b, o_w, rms2_w, gate_w, exp_gate_w, exp_up_w, exp_down_w):
    raise NotImplementedError("write your pallas kernel here")


# speedup vs baseline: 1.3273x; 1.3273x over previous
import functools

import jax
import jax.numpy as jnp
import numpy as np
from jax.experimental import pallas as pl
from jax.experimental.pallas import tpu as pltpu

HID = 896
N_HEADS = 14
KV_HEADS = 2
HEAD_DIM = 64
INTER = 4864
N_EXP = 4
TOP_K = 2
RMS_EPS = 1e-05
S = 2048


ROWB = 256


def _qkv_body(x_ref, w1_ref, qw_ref, qb_ref, kw_ref, kb_ref, vw_ref, vb_ref,
              q_ref, k_ref, v_ref):
    x = x_ref[...]
    var = jnp.mean(jnp.square(x), axis=-1, keepdims=True)
    h = (x * jax.lax.rsqrt(var + RMS_EPS)) * w1_ref[...]
    hb = h.astype(jnp.bfloat16)
    q = jnp.dot(hb, qw_ref[...].astype(jnp.bfloat16),
                preferred_element_type=jnp.float32) + qb_ref[...]
    k = jnp.dot(hb, kw_ref[...].astype(jnp.bfloat16),
                preferred_element_type=jnp.float32) + kb_ref[...]
    v = jnp.dot(hb, vw_ref[...].astype(jnp.bfloat16),
                preferred_element_type=jnp.float32) + vb_ref[...]
    q_ref[...] = q
    k_ref[...] = k
    v_ref[...] = v


def _qkv(x, rms1_w, q_w, q_b, k_w, k_b, v_w, v_b):
    nrb = S // ROWB
    return pl.pallas_call(
        _qkv_body,
        grid=(nrb,),
        in_specs=[
            pl.BlockSpec((ROWB, HID), lambda r: (r, 0)),
            pl.BlockSpec((1, HID), lambda r: (0, 0)),
            pl.BlockSpec((HID, N_HEADS * HEAD_DIM), lambda r: (0, 0)),
            pl.BlockSpec((1, N_HEADS * HEAD_DIM), lambda r: (0, 0)),
            pl.BlockSpec((HID, KV_HEADS * HEAD_DIM), lambda r: (0, 0)),
            pl.BlockSpec((1, KV_HEADS * HEAD_DIM), lambda r: (0, 0)),
            pl.BlockSpec((HID, KV_HEADS * HEAD_DIM), lambda r: (0, 0)),
            pl.BlockSpec((1, KV_HEADS * HEAD_DIM), lambda r: (0, 0)),
        ],
        out_specs=[
            pl.BlockSpec((ROWB, N_HEADS * HEAD_DIM), lambda r: (r, 0)),
            pl.BlockSpec((ROWB, KV_HEADS * HEAD_DIM), lambda r: (r, 0)),
            pl.BlockSpec((ROWB, KV_HEADS * HEAD_DIM), lambda r: (r, 0)),
        ],
        out_shape=[
            jax.ShapeDtypeStruct((S, N_HEADS * HEAD_DIM), jnp.float32),
            jax.ShapeDtypeStruct((S, KV_HEADS * HEAD_DIM), jnp.float32),
            jax.ShapeDtypeStruct((S, KV_HEADS * HEAD_DIM), jnp.float32),
        ],
    )(x, rms1_w, q_w, q_b, k_w, k_b, v_w, v_b)


QB = 512
REP = N_HEADS // KV_HEADS


def _attn_body(q_ref, k_ref, v_ref, o_ref):
    qb = q_ref[0].astype(jnp.bfloat16)
    kb = k_ref[0].astype(jnp.bfloat16)
    s = jax.lax.dot_general(qb, kb, (((1,), (1,)), ((), ())),
                            preferred_element_type=jnp.float32)
    s = s * np.float32(1.0 / np.sqrt(HEAD_DIM))
    m = jnp.max(s, axis=-1, keepdims=True)
    p = jnp.exp(s - m)
    l = jnp.sum(p, axis=-1, keepdims=True)
    pb = (p / l).astype(jnp.bfloat16)
    o_ref[0] = jnp.dot(pb, v_ref[0].astype(jnp.bfloat16),
                       preferred_element_type=jnp.float32)


def _attention(q, k, v):
    nqb = S // QB
    return pl.pallas_call(
        _attn_body,
        grid=(N_HEADS, nqb),
        in_specs=[
            pl.BlockSpec((1, QB, HEAD_DIM), lambda h, r: (h, r, 0)),
            pl.BlockSpec((1, S, HEAD_DIM), lambda h, r: (h // REP, 0, 0)),
            pl.BlockSpec((1, S, HEAD_DIM), lambda h, r: (h // REP, 0, 0)),
        ],
        out_specs=pl.BlockSpec((1, QB, HEAD_DIM), lambda h, r: (h, r, 0)),
        out_shape=jax.ShapeDtypeStruct((N_HEADS, S, HEAD_DIM), jnp.float32),
    )(q, k, v)


def _post_body(attn_ref, res_ref, ow_ref, w2_ref, gw_ref, hid_ref, h2_ref,
               w_ref):
    a = attn_ref[...].astype(jnp.bfloat16)
    ao = jnp.dot(a, ow_ref[...].astype(jnp.bfloat16),
                 preferred_element_type=jnp.float32)
    hid = res_ref[...] + ao
    hid_ref[...] = hid
    var = jnp.mean(jnp.square(hid), axis=-1, keepdims=True)
    h2 = (hid * jax.lax.rsqrt(var + RMS_EPS)) * w2_ref[...]
    h2_ref[...] = h2
    logits = jnp.dot(h2.astype(jnp.bfloat16), gw_ref[...].astype(jnp.bfloat16),
                     preferred_element_type=jnp.float32)[:, :N_EXP]
    lmax = jnp.max(logits, axis=-1, keepdims=True)
    pe = jnp.exp(logits - lmax)
    probs = pe / jnp.sum(pe, axis=-1, keepdims=True)
    li = logits[:, :, None]
    lj = logits[:, None, :]
    gt = (lj > li).astype(jnp.int32)
    eq = (lj == li).astype(jnp.int32)
    tri = (jax.lax.broadcasted_iota(jnp.int32, (1, N_EXP, N_EXP), 2)
           < jax.lax.broadcasted_iota(jnp.int32, (1, N_EXP, N_EXP), 1))
    rank = jnp.sum(gt + eq * tri.astype(jnp.int32), axis=-1)
    sel = (rank < TOP_K).astype(jnp.float32)
    wsel = probs * sel
    w = wsel / jnp.sum(wsel, axis=-1, keepdims=True)
    w_ref[...] = jnp.pad(w, ((0, 0), (0, 128 - N_EXP)))


def _post_attn(attn, x, o_w, rms2_w, gate_w_pad):
    nrb = S // ROWB
    return pl.pallas_call(
        _post_body,
        grid=(nrb,),
        in_specs=[
            pl.BlockSpec((ROWB, N_HEADS * HEAD_DIM), lambda r: (r, 0)),
            pl.BlockSpec((ROWB, HID), lambda r: (r, 0)),
            pl.BlockSpec((N_HEADS * HEAD_DIM, HID), lambda r: (0, 0)),
            pl.BlockSpec((1, HID), lambda r: (0, 0)),
            pl.BlockSpec((HID, 128), lambda r: (0, 0)),
        ],
        out_specs=[
            pl.BlockSpec((ROWB, HID), lambda r: (r, 0)),
            pl.BlockSpec((ROWB, HID), lambda r: (r, 0)),
            pl.BlockSpec((ROWB, 128), lambda r: (r, 0)),
        ],
        out_shape=[
            jax.ShapeDtypeStruct((S, HID), jnp.float32),
            jax.ShapeDtypeStruct((S, HID), jnp.float32),
            jax.ShapeDtypeStruct((S, 128), jnp.float32),
        ],
    )(attn, x, o_w, rms2_w, gate_w_pad)


IB = 256
N_IB = INTER // IB


def _moe_body(h2_ref, hid_ref, w_ref, gw_ref, uw_ref, dw_ref, out_ref,
              acc_ref):
    e = pl.program_id(0)
    i = pl.program_id(1)

    @pl.when((e == 0) & (i == 0))
    def _init():
        acc_ref[...] = hid_ref[...]

    xb = h2_ref[...].astype(jnp.bfloat16)
    g = jnp.dot(xb, gw_ref[0].astype(jnp.bfloat16),
                preferred_element_type=jnp.float32)
    u = jnp.dot(xb, uw_ref[0].astype(jnp.bfloat16),
                preferred_element_type=jnp.float32)
    p = (g / (1.0 + jnp.exp(-g))) * u
    onehot = (jax.lax.broadcasted_iota(jnp.int32, (128, 1), 0) == e
              ).astype(jnp.float32)
    wcol = jnp.dot(w_ref[...], onehot, preferred_element_type=jnp.float32)
    p = (p * wcol).astype(jnp.bfloat16)
    acc_ref[...] += jnp.dot(p, dw_ref[0].astype(jnp.bfloat16),
                            preferred_element_type=jnp.float32)

    @pl.when((e == N_EXP - 1) & (i == N_IB - 1))
    def _fin():
        out_ref[...] = acc_ref[...]


def _moe_dense(h2, hidden, w, exp_gate_w, exp_up_w, exp_down_w):
    return pl.pallas_call(
        _moe_body,
        grid=(N_EXP, N_IB),
        in_specs=[
            pl.BlockSpec((S, HID), lambda e, i: (0, 0)),
            pl.BlockSpec((S, HID), lambda e, i: (0, 0)),
            pl.BlockSpec((S, 128), lambda e, i: (0, 0)),
            pl.BlockSpec((1, HID, IB), lambda e, i: (e, 0, i)),
            pl.BlockSpec((1, HID, IB), lambda e, i: (e, 0, i)),
            pl.BlockSpec((1, IB, HID), lambda e, i: (e, i, 0)),
        ],
        out_specs=pl.BlockSpec((S, HID), lambda e, i: (0, 0)),
        out_shape=jax.ShapeDtypeStruct((S, HID), jnp.float32),
        scratch_shapes=[pltpu.VMEM((S, HID), jnp.float32)],
        compiler_params=pltpu.CompilerParams(
            dimension_semantics=("arbitrary", "arbitrary")),
    )(h2, hidden, w, exp_gate_w, exp_up_w, exp_down_w)


def kernel(hidden_states, position_ids, rms1_w, q_w, q_b, k_w, k_b, v_w, v_b,
           o_w, rms2_w, gate_w, exp_gate_w, exp_up_w, exp_down_w):
    del position_ids
    x = hidden_states.reshape(S, HID)
    q, k, v = _qkv(x, rms1_w.reshape(1, HID), q_w, q_b.reshape(1, -1),
                   k_w, k_b.reshape(1, -1), v_w, v_b.reshape(1, -1))
    q3 = q.reshape(S, N_HEADS, HEAD_DIM).transpose(1, 0, 2)
    k3 = k.reshape(S, KV_HEADS, HEAD_DIM).transpose(1, 0, 2)
    v3 = v.reshape(S, KV_HEADS, HEAD_DIM).transpose(1, 0, 2)
    attn = _attention(q3, k3, v3).transpose(1, 0, 2).reshape(S, N_HEADS * HEAD_DIM)
    gate_w_pad = jnp.pad(gate_w, ((0, 0), (0, 128 - N_EXP)))
    hidden, h2, w = _post_attn(attn, x, o_w, rms2_w.reshape(1, HID),
                               gate_w_pad)
    out = _moe_dense(h2, hidden, w, exp_gate_w, exp_up_w, exp_down_w)
    return out.reshape(1, S, HID)
